# baseline (device time: 177523 ns/iter reference)
import jax
import jax.numpy as jnp
from jax import lax
from jax.experimental import pallas as pl
from jax.experimental.pallas import tpu as pltpu

N_DEV = 16
SQ = 2048
DM = 1024
H_LOC = 8
DH = 128
WINDOW = 128
BAND = 3 * WINDOW
SCALE = 0.08838834764831843

CHUNK = SQ // N_DEV
N_STEPS = 2 * (N_DEV - 1)


def _fused(xb, wq, k, v, wo):

    def body(x_ref, wq_ref, k_ref, v_ref, wo_ref, out_ref,
             stage_ref, recv_ref, send_sems, recv_sems):
        me = lax.axis_index("i")
        right = lax.rem(me + 1, N_DEV)
        left = lax.rem(me + N_DEV - 1, N_DEV)

        def blk(c):
            return pl.ds(lax.rem(c + 2 * N_DEV, N_DEV) * CHUNK, CHUNK)

        def compute_block(c):
            row0 = lax.rem(c + 2 * N_DEV, N_DEV) * CHUNK
            xq = x_ref[pl.ds(row0, CHUNK), :]
            qb = jnp.dot(
                xq, wq_ref[...], preferred_element_type=jnp.float32
            ).astype(jnp.bfloat16)
            t0 = pl.multiple_of(
                jnp.clip((row0 - CHUNK), 0, SQ - BAND), CHUNK
            )
            qi = row0 + lax.broadcasted_iota(jnp.int32, (CHUNK, BAND), 0)
            ki = t0 + lax.broadcasted_iota(jnp.int32, (CHUNK, BAND), 1)
            mask = jnp.abs(qi - ki) <= WINDOW
            ctxs = []
            for h in range(H_LOC):
                qh = qb[:, h * DH:(h + 1) * DH]
                kb = k_ref[pl.ds(t0, BAND), h * DH:(h + 1) * DH]
                vb = v_ref[pl.ds(t0, BAND), h * DH:(h + 1) * DH]
                s = lax.dot_general(
                    qh, kb, (((1,), (1,)), ((), ())),
                    preferred_element_type=jnp.float32,
                ) * SCALE
                w = jnp.exp(jnp.where(mask, s, -1e9))
                w = (w / w.sum(axis=-1, keepdims=True)).astype(jnp.bfloat16)
                ctxs.append(
                    jnp.dot(w, vb, preferred_element_type=jnp.float32)
                )
            ctx = jnp.concatenate(ctxs, axis=1).astype(jnp.bfloat16)
            return jnp.dot(ctx, wo_ref[...], preferred_element_type=jnp.float32)

        def rdma(i, src):
            return pltpu.make_async_remote_copy(
                src_ref=src,
                dst_ref=recv_ref.at[i],
                send_sem=send_sems.at[i],
                recv_sem=recv_sems.at[i],
                device_id=(right,),
                device_id_type=pl.DeviceIdType.MESH,
            )

        barrier_sem = pltpu.get_barrier_semaphore()
        for nbr in (left, right):
            pl.semaphore_signal(
                barrier_sem, inc=1,
                device_id=(nbr,), device_id_type=pl.DeviceIdType.MESH,
            )
        pl.semaphore_wait(barrier_sem, 2)

        descs = {}

        out_ref[blk(me)] = compute_block(me)
        stage_ref[0] = out_ref[blk(me)].astype(jnp.bfloat16)
        descs[0] = rdma(0, stage_ref.at[0])
        descs[0].start()
        out_ref[blk(me - 1)] = compute_block(me - 1)

        for s in range(N_DEV - 1):
            descs[s].wait()
            if s < N_DEV - 2:
                summ = out_ref[blk(me - 1 - s)] + recv_ref[s].astype(
                    jnp.float32
                )
                stage_ref[s + 1] = summ.astype(jnp.bfloat16)
                descs[s + 1] = rdma(s + 1, stage_ref.at[s + 1])
                descs[s + 1].start()
                out_ref[blk(me - 2 - s)] = compute_block(me - 2 - s)
            else:
                summ = out_ref[blk(me + 1)] + recv_ref[s].astype(jnp.float32)
                out_ref[blk(me + 1)] = summ
                stage_ref[N_DEV - 1] = summ.astype(jnp.bfloat16)
                i0 = N_DEV - 1
                descs[i0] = rdma(i0, stage_ref.at[i0])
                descs[i0].start()

        for s in range(N_DEV - 1):
            idx = (N_DEV - 1) + s
            descs[idx].wait()
            if s < N_DEV - 2:
                descs[idx + 1] = rdma(idx + 1, recv_ref.at[idx])
                descs[idx + 1].start()
            out_ref[blk(me - s)] = recv_ref[idx].astype(jnp.float32)

    return pl.pallas_call(
        body,
        out_shape=jax.ShapeDtypeStruct((SQ, DM), jnp.float32),
        in_specs=[pl.BlockSpec(memory_space=pltpu.VMEM)] * 5,
        out_specs=pl.BlockSpec(memory_space=pltpu.VMEM),
        scratch_shapes=[
            pltpu.VMEM((N_DEV, CHUNK, DM), jnp.bfloat16),
            pltpu.VMEM((N_STEPS, CHUNK, DM), jnp.bfloat16),
            pltpu.SemaphoreType.DMA((N_STEPS,)),
            pltpu.SemaphoreType.DMA((N_STEPS,)),
        ],
        compiler_params=pltpu.CompilerParams(collective_id=0),
    )(xb, wq, k, v, wo)


def kernel(x, Wq, K_ext, V_ext, Wo):
    me = lax.axis_index("i")

    xb = x[0].astype(jnp.bfloat16)
    wq = Wq.astype(jnp.bfloat16)
    wo = Wo.astype(jnp.bfloat16)
    k = lax.dynamic_slice_in_dim(K_ext[0], me * H_LOC, H_LOC, axis=1)
    v = lax.dynamic_slice_in_dim(V_ext[0], me * H_LOC, H_LOC, axis=1)
    k = k.astype(jnp.bfloat16).reshape(SQ, H_LOC * DH)
    v = v.astype(jnp.bfloat16).reshape(SQ, H_LOC * DH)

    out = _fused(xb, wq, k, v, wo)
    return out[None]


# device time: 157811 ns/iter; 1.1249x vs baseline; 1.1249x over previous
import jax
import jax.numpy as jnp
from jax import lax
from jax.experimental import pallas as pl
from jax.experimental.pallas import tpu as pltpu

N_DEV = 16
SQ = 2048
DM = 1024
HALF = DM // 2
H_LOC = 8
DH = 128
WINDOW = 128
BAND = 3 * WINDOW
SCALE = 0.08838834764831843

CHUNK = SQ // N_DEV
N_HOPS = N_DEV - 1


def _fused(xb, wq, k, v, wo):

    def body(x_ref, wq_ref, k_ref, v_ref, wo_ref, out_ref,
             stage_ref, recv_ref, agr_stage, agl_stage, agr_recv, agl_recv,
             rs_ssem, rs_rsem, agr_ssem, agr_rsem, agl_ssem, agl_rsem):
        me = lax.axis_index("i")
        right = lax.rem(me + 1, N_DEV)
        left = lax.rem(me + N_DEV - 1, N_DEV)

        def blk(c):
            return pl.ds(lax.rem(c + 2 * N_DEV, N_DEV) * CHUNK, CHUNK)

        def compute_block(c):
            row0 = lax.rem(c + 2 * N_DEV, N_DEV) * CHUNK
            xq = x_ref[pl.ds(row0, CHUNK), :]
            qb = jnp.dot(
                xq, wq_ref[...], preferred_element_type=jnp.float32
            ).astype(jnp.bfloat16)
            t0 = pl.multiple_of(
                jnp.clip(row0 - CHUNK, 0, SQ - BAND), CHUNK
            )
            qi = row0 + lax.broadcasted_iota(jnp.int32, (CHUNK, BAND), 0)
            ki = t0 + lax.broadcasted_iota(jnp.int32, (CHUNK, BAND), 1)
            bias = jnp.where(jnp.abs(qi - ki) <= WINDOW, 0.0, -1e9).astype(
                jnp.float32
            )
            ctxs = []
            for h in range(H_LOC):
                qh = qb[:, h * DH:(h + 1) * DH]
                kb = k_ref[pl.ds(t0, BAND), h * DH:(h + 1) * DH]
                vb = v_ref[pl.ds(t0, BAND), h * DH:(h + 1) * DH]
                s = lax.dot_general(
                    qh, kb, (((1,), (1,)), ((), ())),
                    preferred_element_type=jnp.float32,
                )
                w = jnp.exp(s * SCALE + bias)
                recip = 1.0 / w.sum(axis=-1, keepdims=True)
                ctx_h = jnp.dot(
                    w.astype(jnp.bfloat16), vb,
                    preferred_element_type=jnp.float32,
                )
                ctxs.append(ctx_h * recip)
            ctx = jnp.concatenate(ctxs, axis=1).astype(jnp.bfloat16)
            return jnp.dot(ctx, wo_ref[...], preferred_element_type=jnp.float32)

        def rdma(i, src, dst_slots, ssem, rsem, target):
            return pltpu.make_async_remote_copy(
                src_ref=src,
                dst_ref=dst_slots.at[i],
                send_sem=ssem.at[i],
                recv_sem=rsem.at[i],
                device_id=(target,),
                device_id_type=pl.DeviceIdType.MESH,
            )

        barrier_sem = pltpu.get_barrier_semaphore()
        for nbr in (left, right):
            pl.semaphore_signal(
                barrier_sem, inc=1,
                device_id=(nbr,), device_id_type=pl.DeviceIdType.MESH,
            )
        pl.semaphore_wait(barrier_sem, 2)

        rs = {}

        out_ref[blk(me)] = compute_block(me)
        stage_ref[0] = out_ref[blk(me)].astype(jnp.bfloat16)
        rs[0] = rdma(0, stage_ref.at[0], recv_ref, rs_ssem, rs_rsem, right)
        rs[0].start()
        out_ref[blk(me - 1)] = compute_block(me - 1)

        for s in range(N_HOPS):
            rs[s].wait()
            if s < N_HOPS - 1:
                summ = out_ref[blk(me - 1 - s)] + recv_ref[s].astype(
                    jnp.float32
                )
                stage_ref[s + 1] = summ.astype(jnp.bfloat16)
                rs[s + 1] = rdma(
                    s + 1, stage_ref.at[s + 1], recv_ref, rs_ssem, rs_rsem,
                    right,
                )
                rs[s + 1].start()
                out_ref[blk(me - 2 - s)] = compute_block(me - 2 - s)
            else:
                summ = out_ref[blk(me + 1)] + recv_ref[s].astype(jnp.float32)
                out_ref[blk(me + 1)] = summ
                sb = summ.astype(jnp.bfloat16)
                agr_stage[...] = sb[:, :HALF]
                agl_stage[...] = sb[:, HALF:]

        agr = {0: rdma(0, agr_stage, agr_recv, agr_ssem, agr_rsem, right)}
        agl = {0: rdma(0, agl_stage, agl_recv, agl_ssem, agl_rsem, left)}
        agr[0].start()
        agl[0].start()

        for s in range(N_HOPS):
            agr[s].wait()
            if s < N_HOPS - 1:
                agr[s + 1] = rdma(
                    s + 1, agr_recv.at[s], agr_recv, agr_ssem, agr_rsem, right
                )
                agr[s + 1].start()
            agl[s].wait()
            if s < N_HOPS - 1:
                agl[s + 1] = rdma(
                    s + 1, agl_recv.at[s], agl_recv, agl_ssem, agl_rsem, left
                )
                agl[s + 1].start()
            out_ref[blk(me - s), 0:HALF] = agr_recv[s].astype(jnp.float32)
            out_ref[blk(me + 2 + s), HALF:DM] = agl_recv[s].astype(jnp.float32)

    return pl.pallas_call(
        body,
        out_shape=jax.ShapeDtypeStruct((SQ, DM), jnp.float32),
        in_specs=[pl.BlockSpec(memory_space=pltpu.VMEM)] * 5,
        out_specs=pl.BlockSpec(memory_space=pltpu.VMEM),
        scratch_shapes=[
            pltpu.VMEM((N_HOPS, CHUNK, DM), jnp.bfloat16),
            pltpu.VMEM((N_HOPS, CHUNK, DM), jnp.bfloat16),
            pltpu.VMEM((CHUNK, HALF), jnp.bfloat16),
            pltpu.VMEM((CHUNK, HALF), jnp.bfloat16),
            pltpu.VMEM((N_HOPS, CHUNK, HALF), jnp.bfloat16),
            pltpu.VMEM((N_HOPS, CHUNK, HALF), jnp.bfloat16),
            pltpu.SemaphoreType.DMA((N_HOPS,)),
            pltpu.SemaphoreType.DMA((N_HOPS,)),
            pltpu.SemaphoreType.DMA((N_HOPS,)),
            pltpu.SemaphoreType.DMA((N_HOPS,)),
            pltpu.SemaphoreType.DMA((N_HOPS,)),
            pltpu.SemaphoreType.DMA((N_HOPS,)),
        ],
        compiler_params=pltpu.CompilerParams(collective_id=0),
    )(xb, wq, k, v, wo)


def kernel(x, Wq, K_ext, V_ext, Wo):
    me = lax.axis_index("i")

    xb = x[0].astype(jnp.bfloat16)
    wq = Wq.astype(jnp.bfloat16)
    wo = Wo.astype(jnp.bfloat16)
    k = lax.dynamic_slice_in_dim(K_ext[0], me * H_LOC, H_LOC, axis=1)
    v = lax.dynamic_slice_in_dim(V_ext[0], me * H_LOC, H_LOC, axis=1)
    k = k.astype(jnp.bfloat16).reshape(SQ, H_LOC * DH)
    v = v.astype(jnp.bfloat16).reshape(SQ, H_LOC * DH)

    out = _fused(xb, wq, k, v, wo)
    return out[None]


# device time: 144978 ns/iter; 1.2245x vs baseline; 1.0885x over previous
import jax
import jax.numpy as jnp
from jax import lax
from jax.experimental import pallas as pl
from jax.experimental.pallas import tpu as pltpu

N_DEV = 16
SQ = 2048
DM = 1024
H_LOC = 8
DH = 128
WINDOW = 128
BAND = 3 * WINDOW
SCALE = 0.08838834764831843

CHUNK = SQ // N_DEV
N_HOPS = N_DEV - 1
AGR_HOPS = 8
AGL_HOPS = 7


def _fused(xb, wq, k, v, wo):

    def body(x_ref, wq_ref, k_ref, v_ref, wo_ref, out_ref,
             stage_ref, recv_ref,
             rs_ssem, rs_rsem, agr_ssem, agr_rsem, agl_ssem, agl_rsem):
        me = lax.axis_index("i")
        right = lax.rem(me + 1, N_DEV)
        left = lax.rem(me + N_DEV - 1, N_DEV)

        def blk(c):
            return pl.ds(lax.rem(c + 2 * N_DEV, N_DEV) * CHUNK, CHUNK)

        def compute_block(c):
            row0 = lax.rem(c + 2 * N_DEV, N_DEV) * CHUNK
            xq = x_ref[pl.ds(row0, CHUNK), :]
            qb = jnp.dot(
                xq, wq_ref[...], preferred_element_type=jnp.float32
            ).astype(jnp.bfloat16)
            t0 = pl.multiple_of(
                jnp.clip(row0 - CHUNK, 0, SQ - BAND), CHUNK
            )
            qi = row0 + lax.broadcasted_iota(jnp.int32, (CHUNK, BAND), 0)
            ki = t0 + lax.broadcasted_iota(jnp.int32, (CHUNK, BAND), 1)
            bias = jnp.where(jnp.abs(qi - ki) <= WINDOW, 0.0, -1e9).astype(
                jnp.float32
            )
            ctxs = []
            for h in range(H_LOC):
                qh = qb[:, h * DH:(h + 1) * DH]
                kb = k_ref[pl.ds(t0, BAND), h * DH:(h + 1) * DH]
                vb = v_ref[pl.ds(t0, BAND), h * DH:(h + 1) * DH]
                s = lax.dot_general(
                    qh, kb, (((1,), (1,)), ((), ())),
                    preferred_element_type=jnp.float32,
                )
                w = jnp.exp(s * SCALE + bias)
                recip = 1.0 / w.sum(axis=-1, keepdims=True)
                ctx_h = jnp.dot(
                    w.astype(jnp.bfloat16), vb,
                    preferred_element_type=jnp.float32,
                )
                ctxs.append(ctx_h * recip)
            ctx = jnp.concatenate(ctxs, axis=1).astype(jnp.bfloat16)
            return jnp.dot(ctx, wo_ref[...], preferred_element_type=jnp.float32)

        def rs_rdma(i):
            return pltpu.make_async_remote_copy(
                src_ref=stage_ref.at[i],
                dst_ref=recv_ref.at[i],
                send_sem=rs_ssem.at[i],
                recv_sem=rs_rsem.at[i],
                device_id=(right,),
                device_id_type=pl.DeviceIdType.MESH,
            )

        def ag_rdma(i, chunk, ssem, rsem, target):
            return pltpu.make_async_remote_copy(
                src_ref=out_ref.at[blk(chunk)],
                dst_ref=out_ref.at[blk(chunk)],
                send_sem=ssem.at[i],
                recv_sem=rsem.at[i],
                device_id=(target,),
                device_id_type=pl.DeviceIdType.MESH,
            )

        barrier_sem = pltpu.get_barrier_semaphore()
        for nbr in (left, right):
            pl.semaphore_signal(
                barrier_sem, inc=1,
                device_id=(nbr,), device_id_type=pl.DeviceIdType.MESH,
            )
        pl.semaphore_wait(barrier_sem, 2)

        rs = {}
        stage_ref[0] = compute_block(me).astype(jnp.bfloat16)
        rs[0] = rs_rdma(0)
        rs[0].start()
        pend = compute_block(me - 1)

        for s in range(N_HOPS):
            rs[s].wait()
            if s < N_HOPS - 1:
                summ = pend + recv_ref[s].astype(jnp.float32)
                stage_ref[s + 1] = summ.astype(jnp.bfloat16)
                rs[s + 1] = rs_rdma(s + 1)
                rs[s + 1].start()
                pend = compute_block(me - 2 - s)
            else:
                summ = pend + recv_ref[s].astype(jnp.float32)
                out_ref[blk(me + 1)] = summ.astype(jnp.bfloat16)

        agr = {0: ag_rdma(0, me + 1, agr_ssem, agr_rsem, right)}
        agl = {0: ag_rdma(0, me + 1, agl_ssem, agl_rsem, left)}
        agr[0].start()
        agl[0].start()

        for s in range(AGR_HOPS):
            agr[s].wait()
            if s < AGR_HOPS - 1:
                agr[s + 1] = ag_rdma(s + 1, me - s, agr_ssem, agr_rsem, right)
                agr[s + 1].start()
            if s < AGL_HOPS:
                agl[s].wait()
                if s < AGL_HOPS - 1:
                    agl[s + 1] = ag_rdma(
                        s + 1, me + 2 + s, agl_ssem, agl_rsem, left
                    )
                    agl[s + 1].start()

    return pl.pallas_call(
        body,
        out_shape=jax.ShapeDtypeStruct((SQ, DM), jnp.bfloat16),
        in_specs=[pl.BlockSpec(memory_space=pltpu.VMEM)] * 5,
        out_specs=pl.BlockSpec(memory_space=pltpu.VMEM),
        scratch_shapes=[
            pltpu.VMEM((N_HOPS, CHUNK, DM), jnp.bfloat16),
            pltpu.VMEM((N_HOPS, CHUNK, DM), jnp.bfloat16),
            pltpu.SemaphoreType.DMA((N_HOPS,)),
            pltpu.SemaphoreType.DMA((N_HOPS,)),
            pltpu.SemaphoreType.DMA((AGR_HOPS,)),
            pltpu.SemaphoreType.DMA((AGR_HOPS,)),
            pltpu.SemaphoreType.DMA((AGL_HOPS,)),
            pltpu.SemaphoreType.DMA((AGL_HOPS,)),
        ],
        compiler_params=pltpu.CompilerParams(collective_id=0),
    )(xb, wq, k, v, wo)


def kernel(x, Wq, K_ext, V_ext, Wo):
    me = lax.axis_index("i")

    xb = x[0].astype(jnp.bfloat16)
    wq = Wq.astype(jnp.bfloat16)
    wo = Wo.astype(jnp.bfloat16)
    k = lax.dynamic_slice_in_dim(K_ext[0], me * H_LOC, H_LOC, axis=1)
    v = lax.dynamic_slice_in_dim(V_ext[0], me * H_LOC, H_LOC, axis=1)
    k = k.astype(jnp.bfloat16).reshape(SQ, H_LOC * DH)
    v = v.astype(jnp.bfloat16).reshape(SQ, H_LOC * DH)

    out = _fused(xb, wq, k, v, wo)
    return out[None]


# device time: 138014 ns/iter; 1.2863x vs baseline; 1.0505x over previous
import jax
import jax.numpy as jnp
from jax import lax
from jax.experimental import pallas as pl
from jax.experimental.pallas import tpu as pltpu

N_DEV = 16
SQ = 2048
DM = 1024
H_LOC = 8
DH = 128
WINDOW = 128
BAND = 3 * WINDOW
SCALE = 0.08838834764831843

CHUNK = SQ // N_DEV
N_HOPS = N_DEV - 1
AGR_HOPS = 8
AGL_HOPS = 7


def _fused(xb, wq, k, v, wo):

    def body(x_ref, wq_ref, k_ref, v_ref, wo_ref, out_ref,
             stage_ref, recv_ref,
             rs_ssem, rs_rsem, agr_ssem, agr_rsem, agl_ssem, agl_rsem):
        me = lax.axis_index("i")
        right = lax.rem(me + 1, N_DEV)
        left = lax.rem(me + N_DEV - 1, N_DEV)

        def blk(c):
            return pl.ds(lax.rem(c + 2 * N_DEV, N_DEV) * CHUNK, CHUNK)

        def compute_block(c):
            row0 = lax.rem(c + 2 * N_DEV, N_DEV) * CHUNK
            xq = x_ref[pl.ds(row0, CHUNK), :].astype(jnp.bfloat16)
            qb = jnp.dot(
                xq, wq_ref[...], preferred_element_type=jnp.float32
            ).astype(jnp.bfloat16)
            t0 = pl.multiple_of(
                jnp.clip(row0 - CHUNK, 0, SQ - BAND), CHUNK
            )
            qi = row0 + lax.broadcasted_iota(jnp.int32, (CHUNK, BAND), 0)
            ki = t0 + lax.broadcasted_iota(jnp.int32, (CHUNK, BAND), 1)
            bias = jnp.where(jnp.abs(qi - ki) <= WINDOW, 0.0, -1e9).astype(
                jnp.bfloat16
            )
            scale = jnp.bfloat16(SCALE)
            ctxs = []
            for h in range(H_LOC):
                qh = qb[:, h * DH:(h + 1) * DH]
                kb = k_ref[pl.ds(t0, BAND), h * DH:(h + 1) * DH]
                vb = v_ref[pl.ds(t0, BAND), h * DH:(h + 1) * DH]
                s = lax.dot_general(
                    qh, kb, (((1,), (1,)), ((), ())),
                    preferred_element_type=jnp.float32,
                ).astype(jnp.bfloat16)
                w = jnp.exp(s * scale + bias)
                denom = jnp.sum(
                    w, axis=-1, keepdims=True, dtype=jnp.float32
                )
                recip = 1.0 / denom
                ctx_h = jnp.dot(
                    w, vb, preferred_element_type=jnp.float32
                )
                ctxs.append((ctx_h * recip).astype(jnp.bfloat16))
            ctx = jnp.concatenate(ctxs, axis=1)
            return jnp.dot(
                ctx, wo_ref[...], preferred_element_type=jnp.float32
            ).astype(jnp.bfloat16)

        def rs_rdma(i):
            return pltpu.make_async_remote_copy(
                src_ref=stage_ref.at[i],
                dst_ref=recv_ref.at[i],
                send_sem=rs_ssem.at[i],
                recv_sem=rs_rsem.at[i],
                device_id=(right,),
                device_id_type=pl.DeviceIdType.MESH,
            )

        def ag_rdma(i, chunk, ssem, rsem, target):
            return pltpu.make_async_remote_copy(
                src_ref=out_ref.at[blk(chunk)],
                dst_ref=out_ref.at[blk(chunk)],
                send_sem=ssem.at[i],
                recv_sem=rsem.at[i],
                device_id=(target,),
                device_id_type=pl.DeviceIdType.MESH,
            )

        barrier_sem = pltpu.get_barrier_semaphore()
        for nbr in (left, right):
            pl.semaphore_signal(
                barrier_sem, inc=1,
                device_id=(nbr,), device_id_type=pl.DeviceIdType.MESH,
            )
        pl.semaphore_wait(barrier_sem, 2)

        rs = {}
        stage_ref[0] = compute_block(me)
        rs[0] = rs_rdma(0)
        rs[0].start()
        pend = compute_block(me - 1)

        for s in range(N_HOPS):
            rs[s].wait()
            if s < N_HOPS - 1:
                stage_ref[s + 1] = pend + recv_ref[s]
                rs[s + 1] = rs_rdma(s + 1)
                rs[s + 1].start()
                pend = compute_block(me - 2 - s)
            else:
                out_ref[blk(me + 1)] = pend + recv_ref[s]

        agr = {0: ag_rdma(0, me + 1, agr_ssem, agr_rsem, right)}
        agl = {0: ag_rdma(0, me + 1, agl_ssem, agl_rsem, left)}
        agr[0].start()
        agl[0].start()

        for s in range(AGR_HOPS):
            agr[s].wait()
            if s < AGR_HOPS - 1:
                agr[s + 1] = ag_rdma(s + 1, me - s, agr_ssem, agr_rsem, right)
                agr[s + 1].start()
            if s < AGL_HOPS:
                agl[s].wait()
                if s < AGL_HOPS - 1:
                    agl[s + 1] = ag_rdma(
                        s + 1, me + 2 + s, agl_ssem, agl_rsem, left
                    )
                    agl[s + 1].start()

    return pl.pallas_call(
        body,
        out_shape=jax.ShapeDtypeStruct((SQ, DM), jnp.bfloat16),
        in_specs=[pl.BlockSpec(memory_space=pltpu.VMEM)] * 5,
        out_specs=pl.BlockSpec(memory_space=pltpu.VMEM),
        scratch_shapes=[
            pltpu.VMEM((N_HOPS, CHUNK, DM), jnp.bfloat16),
            pltpu.VMEM((N_HOPS, CHUNK, DM), jnp.bfloat16),
            pltpu.SemaphoreType.DMA((N_HOPS,)),
            pltpu.SemaphoreType.DMA((N_HOPS,)),
            pltpu.SemaphoreType.DMA((AGR_HOPS,)),
            pltpu.SemaphoreType.DMA((AGR_HOPS,)),
            pltpu.SemaphoreType.DMA((AGL_HOPS,)),
            pltpu.SemaphoreType.DMA((AGL_HOPS,)),
        ],
        compiler_params=pltpu.CompilerParams(collective_id=0),
    )(xb, wq, k, v, wo)


def kernel(x, Wq, K_ext, V_ext, Wo):
    me = lax.axis_index("i")

    xb = x[0]
    wq = Wq.astype(jnp.bfloat16)
    wo = Wo.astype(jnp.bfloat16)
    k = lax.dynamic_slice_in_dim(K_ext[0], me * H_LOC, H_LOC, axis=1)
    v = lax.dynamic_slice_in_dim(V_ext[0], me * H_LOC, H_LOC, axis=1)
    k = k.astype(jnp.bfloat16).reshape(SQ, H_LOC * DH)
    v = v.astype(jnp.bfloat16).reshape(SQ, H_LOC * DH)

    out = _fused(xb, wq, k, v, wo)
    return out[None]
